# baseline (speedup 1.0000x reference)
import jax
import jax.numpy as jnp
from jax import lax
from jax.experimental import pallas as pl
from jax.experimental.pallas import tpu as pltpu
from jax.experimental.pallas import tpu_sc as plsc

NC, NS, L = 2, 16, 16
NW = NC * NS
B, S, D = 4, 2048, 1024
ROWS = B * S
RPW = ROWS // NW
WPB = S // RPW
CH = 16
NCH = RPW // CH
VPR = D // L
NBUF = 3
NGRP = NCH // NBUF
TAIL = NCH - NGRP * NBUF
SKEW = 2


def _sc_body(x_hbm, idx_hbm, pe_hbm, out_hbm, idx_v, *bufs_and_sems):
    x_v = bufs_and_sems[0:NBUF]
    pe_v = bufs_and_sems[NBUF : 2 * NBUF]
    xsem = bufs_and_sems[2 * NBUF : 3 * NBUF]
    gsem = bufs_and_sems[3 * NBUF : 4 * NBUF]
    osem = bufs_and_sems[4 * NBUF : 5 * NBUF]

    wid = lax.axis_index("s") * NC + lax.axis_index("c")
    bb = wid // WPB
    s0 = (wid % WPB) * RPW
    pltpu.sync_copy(idx_hbm.at[bb, pl.ds(s0, RPW)], idx_v)

    def issue_in(c, b):
        pltpu.async_copy(x_hbm.at[bb, pl.ds(s0 + c * CH, CH)], x_v[b], xsem[b])
        pltpu.async_copy(pe_hbm.at[idx_v.at[pl.ds(c * CH, CH)]], pe_v[b], gsem[b])

    def wait_in(b):
        pltpu.make_async_copy(x_hbm.at[0, pl.ds(0, CH)], x_v[b], xsem[b]).wait()
        pltpu.make_async_copy(
            pe_hbm.at[idx_v.at[pl.ds(0, CH)]], pe_v[b], gsem[b]
        ).wait()

    def wait_out(b):
        pltpu.make_async_copy(out_hbm.at[0, pl.ds(0, CH)], x_v[b], osem[b]).wait()

    def chunk_step(c, b, refill):
        wait_in(b)
        if refill:
            b2 = (b + SKEW) % NBUF

            @pl.when(jnp.logical_and(c >= NBUF - SKEW, c <= NCH - 1 - SKEW))
            def _refill():
                wait_out(b2)
                issue_in(c + SKEW, b2)

        xb = x_v[b]
        pb = pe_v[b]

        @plsc.parallel_loop(0, CH * VPR, 1, unroll=8)
        def _add(i):
            r = i // VPR
            j = (i % VPR) * L
            plsc.addupdate(xb.at[r, pl.ds(j, L)], pb[r, pl.ds(j, L)])

        pltpu.async_copy(
            x_v[b], out_hbm.at[bb, pl.ds(s0 + c * CH, CH)], osem[b]
        )

    for b in range(NBUF):
        issue_in(b, b)

    def group(g, carry):
        for b in range(NBUF):
            chunk_step(g * NBUF + b, b, True)
        return carry

    lax.fori_loop(0, NGRP, group, 0)

    for t in range(TAIL):
        chunk_step(NGRP * NBUF + t, (NGRP * NBUF + t) % NBUF, False)

    for b in range(NBUF):
        wait_out(b)


@jax.jit
def _sc_call(x, rel_times, pe):
    mesh = plsc.VectorSubcoreMesh(
        core_axis_name="c", subcore_axis_name="s", num_cores=NC, num_subcores=NS
    )
    scratch = (
        [pltpu.VMEM((RPW,), jnp.int32)]
        + [pltpu.VMEM((CH, D), jnp.float32) for _ in range(2 * NBUF)]
        + [pltpu.SemaphoreType.DMA for _ in range(3 * NBUF)]
    )
    return pl.kernel(
        _sc_body,
        out_type=jax.ShapeDtypeStruct((B, S, D), jnp.float32),
        mesh=mesh,
        scratch_types=scratch,
    )(x, rel_times, pe)


def kernel(x, rel_times, pe):
    return _sc_call(x, rel_times.astype(jnp.int32), pe)


# speedup vs baseline: 1.0337x; 1.0337x over previous
import jax
import jax.numpy as jnp
from jax import lax
from jax.experimental import pallas as pl
from jax.experimental.pallas import tpu as pltpu
from jax.experimental.pallas import tpu_sc as plsc

NC, NS, L = 2, 16, 16
NW = NC * NS
B, S, D = 4, 2048, 1024
ROWS = B * S
RPW = ROWS // NW
WPB = S // RPW
CH = 8
NCH = RPW // CH
VPR = D // L
NBUF = 4
NGRP = NCH // NBUF
SKEW = 3


def _sc_body(x_hbm, idx_hbm, pe_hbm, out_hbm, idx_v, *bufs_and_sems):
    x_v = bufs_and_sems[0:NBUF]
    pe_v = bufs_and_sems[NBUF : 2 * NBUF]
    xsem = bufs_and_sems[2 * NBUF : 3 * NBUF]
    gsem = bufs_and_sems[3 * NBUF : 4 * NBUF]
    osem = bufs_and_sems[4 * NBUF : 5 * NBUF]

    wid = lax.axis_index("s") * NC + lax.axis_index("c")
    bb = wid // WPB
    s0 = (wid % WPB) * RPW
    pltpu.sync_copy(idx_hbm.at[bb, pl.ds(s0, RPW)], idx_v)

    def issue_in(c, b):
        pltpu.async_copy(x_hbm.at[bb, pl.ds(s0 + c * CH, CH)], x_v[b], xsem[b])
        pltpu.async_copy(pe_hbm.at[idx_v.at[pl.ds(c * CH, CH)]], pe_v[b], gsem[b])

    def wait_in(b):
        pltpu.make_async_copy(x_hbm.at[0, pl.ds(0, CH)], x_v[b], xsem[b]).wait()
        pltpu.make_async_copy(
            pe_hbm.at[idx_v.at[pl.ds(0, CH)]], pe_v[b], gsem[b]
        ).wait()

    def wait_out(b):
        pltpu.make_async_copy(out_hbm.at[0, pl.ds(0, CH)], x_v[b], osem[b]).wait()

    for b in range(NBUF):
        issue_in(b, b)

    def group(g, carry):
        for b in range(NBUF):
            c = g * NBUF + b
            b2 = (b + SKEW) % NBUF

            @pl.when(jnp.logical_and(c >= NBUF - SKEW, c <= NCH - 1 - SKEW))
            def _refill():
                wait_out(b2)
                issue_in(c + SKEW, b2)

            wait_in(b)

            xb = x_v[b]
            pb = pe_v[b]

            @plsc.parallel_loop(0, CH * VPR, 1, unroll=8)
            def _add(i):
                r = i // VPR
                j = (i % VPR) * L
                plsc.addupdate(xb.at[r, pl.ds(j, L)], pb[r, pl.ds(j, L)])

            pltpu.async_copy(
                x_v[b], out_hbm.at[bb, pl.ds(s0 + c * CH, CH)], osem[b]
            )
        return carry

    lax.fori_loop(0, NGRP, group, 0)

    for b in range(NBUF):
        wait_out(b)


@jax.jit
def _sc_call(x, rel_times, pe):
    mesh = plsc.VectorSubcoreMesh(
        core_axis_name="c", subcore_axis_name="s", num_cores=NC, num_subcores=NS
    )
    scratch = (
        [pltpu.VMEM((RPW,), jnp.int32)]
        + [pltpu.VMEM((CH, D), jnp.float32) for _ in range(2 * NBUF)]
        + [pltpu.SemaphoreType.DMA for _ in range(3 * NBUF)]
    )
    return pl.kernel(
        _sc_body,
        out_type=jax.ShapeDtypeStruct((B, S, D), jnp.float32),
        mesh=mesh,
        scratch_types=scratch,
    )(x, rel_times, pe)


def kernel(x, rel_times, pe):
    return _sc_call(x, rel_times.astype(jnp.int32), pe)
